# Initial kernel scaffold; baseline (speedup 1.0000x reference)
import jax
import jax.numpy as jnp
from jax.experimental import pallas as pl


def kernel(node_attr, edge_attr, edge_sh, W1, b1, W2, b2, edge_index):
    raise NotImplementedError("write your pallas kernel here")


# speedup vs baseline: 1.4486x; 1.4486x over previous
import functools

import numpy as np
import jax
import jax.numpy as jnp
from jax import lax
from jax.experimental import pallas as pl
from jax.experimental.pallas import tpu as pltpu
from jax.experimental.pallas import tpu_sc as plsc

NS = 16
NV = 4
SH = 9
WN = NS * NS + NS * NV
TP_W = 32

NW = 32
CH = 125
VB = 500
BE = 2000


def _sel_matrices():
    r = np.zeros((NS, WN), np.float32)
    for c in range(NS * NS):
        r[c // NS, c] = 1.0
    for c in range(NS * NV):
        r[c // NV, NS * NS + c] = 1.0
    s = np.zeros((WN, TP_W), np.float32)
    norm = 1.0 / np.sqrt(float(NS))
    for i in range(NS):
        for m in range(NS):
            s[i * NS + m, m] = norm
        for m in range(NV):
            for k in range(3):
                s[NS * NS + i * NV + m, NS + m * 3 + k] = norm
    q = np.zeros((SH, TP_W), np.float32)
    q[0, :NS] = 1.0
    for m in range(NV):
        for k in range(3):
            q[1 + k, NS + m * 3 + k] = 1.0
    return jnp.asarray(r), jnp.asarray(s), jnp.asarray(q)


def _gather_body(node_hbm, dst_hbm, x_hbm, idx_v, rows_v, sem):
    wid = lax.axis_index("s") * 2 + lax.axis_index("c")
    pltpu.sync_copy(dst_hbm.at[wid], idx_v)
    nch = idx_v.shape[0]

    def body(j, carry):
        pltpu.async_copy(node_hbm.at[idx_v.at[j]],
                         rows_v.at[pl.ds(j * CH, CH)], sem).wait()
        return carry

    lax.fori_loop(0, nch, body, 0)
    pltpu.sync_copy(rows_v, x_hbm.at[wid])


def _gather(node_attr, dst3):
    n_nodes = node_attr.shape[0]
    epw = dst3.shape[1] * dst3.shape[2]
    mesh = plsc.VectorSubcoreMesh(core_axis_name="c", subcore_axis_name="s")
    k = functools.partial(
        pl.kernel,
        out_type=jax.ShapeDtypeStruct((NW, epw, NS), jnp.float32),
        mesh=mesh,
        compiler_params=pltpu.CompilerParams(use_tc_tiling_on_sc=False),
        scratch_types=[
            pltpu.VMEM(dst3.shape[1:], jnp.int32),
            pltpu.VMEM((epw, NS), jnp.float32),
            pltpu.SemaphoreType.DMA,
        ],
    )(_gather_body)
    return k(node_attr, dst3)


def _scatter_body(tp_hbm, src_hbm, out_hbm, idx_v, vals_v, zrow_v, acc_sh):
    cid = lax.axis_index("c")
    sid = lax.axis_index("s")
    wid = sid * 2 + cid
    stripe = acc_sh.shape[0] // 16

    def zb(j, carry):
        zrow_v[j, pl.ds(0, 16)] = jnp.zeros((16,), jnp.float32)
        zrow_v[j, pl.ds(16, 16)] = jnp.zeros((16,), jnp.float32)
        return carry

    lax.fori_loop(0, stripe, zb, 0)
    pltpu.sync_copy(zrow_v, acc_sh.at[pl.ds(sid * stripe, stripe)])
    plsc.subcore_barrier()

    pltpu.sync_copy(src_hbm.at[wid], idx_v)
    nvb = tp_hbm.shape[1]

    def body(cc, carry):
        pltpu.sync_copy(tp_hbm.at[wid, cc], vals_v)
        for kk in range(VB // CH):
            pltpu.sync_copy(vals_v.at[pl.ds(kk * CH, CH)],
                            acc_sh.at[idx_v.at[cc * (VB // CH) + kk]],
                            add=True)
        return carry

    lax.fori_loop(0, nvb, body, 0)
    plsc.subcore_barrier()
    pltpu.sync_copy(acc_sh.at[pl.ds(sid * stripe, stripe)],
                    out_hbm.at[cid, pl.ds(sid * stripe, stripe)])


def _scatter(tp4, src3, n_nodes):
    mesh = plsc.VectorSubcoreMesh(core_axis_name="c", subcore_axis_name="s")
    k = functools.partial(
        pl.kernel,
        out_type=jax.ShapeDtypeStruct((2, n_nodes, TP_W), jnp.float32),
        mesh=mesh,
        compiler_params=pltpu.CompilerParams(use_tc_tiling_on_sc=False),
        scratch_types=[
            pltpu.VMEM(src3.shape[1:], jnp.int32),
            pltpu.VMEM((VB, TP_W), jnp.float32),
            pltpu.VMEM((n_nodes // 16, TP_W), jnp.float32),
            pltpu.VMEM_SHARED((n_nodes, TP_W), jnp.float32),
        ],
    )(_scatter_body)
    return k(tp4, src3)


def _tp_body(ea_ref, x_ref, sh_ref, w1_ref, b1_ref, w2_ref, b2_ref,
             r_ref, s_ref, q_ref, out_ref):
    hp = lax.Precision.HIGHEST
    h = jnp.maximum(
        jnp.dot(ea_ref[...], w1_ref[...], precision=hp,
                preferred_element_type=jnp.float32) + b1_ref[...], 0.0)
    w = jnp.dot(h, w2_ref[...], precision=hp,
                preferred_element_type=jnp.float32) + b2_ref[...]
    xr = jnp.dot(x_ref[...], r_ref[...], precision=hp,
                 preferred_element_type=jnp.float32)
    a = jnp.dot(xr * w, s_ref[...], precision=hp,
                preferred_element_type=jnp.float32)
    shx = jnp.dot(sh_ref[...], q_ref[...], precision=hp,
                  preferred_element_type=jnp.float32)
    lane = lax.broadcasted_iota(jnp.int32, out_ref.shape, 1)
    out_ref[...] = a * shx + jnp.where(lane == NS + NV * 3, 1.0, 0.0)


def _tp_edges(edge_attr, x, edge_sh, W1, b1, W2, b2, R, S, Q):
    e = edge_attr.shape[0]
    grid = e // BE
    full = lambda i: (0, 0)
    blk = lambda i: (i, 0)
    return pl.pallas_call(
        _tp_body,
        grid=(grid,),
        in_specs=[
            pl.BlockSpec((BE, edge_attr.shape[1]), blk),
            pl.BlockSpec((BE, NS), blk),
            pl.BlockSpec((BE, SH), blk),
            pl.BlockSpec(W1.shape, full),
            pl.BlockSpec((1, b1.shape[1]), full),
            pl.BlockSpec(W2.shape, full),
            pl.BlockSpec((1, b2.shape[1]), full),
            pl.BlockSpec(R.shape, full),
            pl.BlockSpec(S.shape, full),
            pl.BlockSpec(Q.shape, full),
        ],
        out_specs=pl.BlockSpec((BE, TP_W), blk),
        out_shape=jax.ShapeDtypeStruct((e, TP_W), jnp.float32),
    )(edge_attr, x, edge_sh, W1, b1, W2, b2, R, S, Q)


def _combine_body(p_ref, o_ref):
    ps = p_ref[0] + p_ref[1]
    cnt = jnp.maximum(ps[:, NS + NV * 3:NS + NV * 3 + 1], 1.0)
    o_ref[...] = ps[:, :NS + NV * 3] / cnt


def _combine(partials):
    n = partials.shape[1]
    return pl.pallas_call(
        _combine_body,
        grid=(1,),
        in_specs=[pl.BlockSpec(partials.shape, lambda i: (0, 0, 0))],
        out_specs=pl.BlockSpec((n, NS + NV * 3), lambda i: (0, 0)),
        out_shape=jax.ShapeDtypeStruct((n, NS + NV * 3), jnp.float32),
    )(partials)


def kernel(node_attr, edge_attr, edge_sh, W1, b1, W2, b2, edge_index):
    n_nodes = node_attr.shape[0]
    e = edge_attr.shape[0]
    epw = e // NW
    R, S, Q = _sel_matrices()

    src3 = edge_index[0].reshape(NW, epw // CH, CH)
    dst3 = edge_index[1].reshape(NW, epw // CH, CH)

    x = _gather(node_attr, dst3).reshape(e, NS)
    tp = _tp_edges(edge_attr, x, edge_sh, W1, b1.reshape(1, -1),
                   W2, b2.reshape(1, -1), R, S, Q)
    partials = _scatter(tp.reshape(NW, epw // VB, VB, TP_W), src3, n_nodes)
    return _combine(partials)


# speedup vs baseline: 7.9161x; 5.4647x over previous
import functools

import numpy as np
import jax
import jax.numpy as jnp
from jax import lax
from jax.experimental import pallas as pl
from jax.experimental.pallas import tpu as pltpu
from jax.experimental.pallas import tpu_sc as plsc

NS = 16
NV = 4
SH = 9
WN = NS * NS + NS * NV
TP_W = 32

NW = 32
CH = 40
VB = 1000
BE = 6400


def _sel_matrices():
    r = np.zeros((NS, WN), np.float32)
    for c in range(NS * NS):
        r[c // NS, c] = 1.0
    for c in range(NS * NV):
        r[c // NV, NS * NS + c] = 1.0
    s = np.zeros((WN, TP_W), np.float32)
    norm = 1.0 / np.sqrt(float(NS))
    for i in range(NS):
        for m in range(NS):
            s[i * NS + m, m] = norm
        for m in range(NV):
            for k in range(3):
                s[NS * NS + i * NV + m, NS + m * 3 + k] = norm
    q = np.zeros((SH, TP_W), np.float32)
    q[0, :NS] = 1.0
    for m in range(NV):
        for k in range(3):
            q[1 + k, NS + m * 3 + k] = 1.0
    return jnp.asarray(r), jnp.asarray(s), jnp.asarray(q)


def _gather_body(node_hbm, dst_hbm, x_hbm, idx_v, rows_v, sem):
    wid = lax.axis_index("s") * 2 + lax.axis_index("c")
    epw = idx_v.shape[0]
    base = wid * epw
    pltpu.sync_copy(dst_hbm.at[pl.ds(base, epw)], idx_v)
    nch = epw // CH

    def fire(j, carry):
        pltpu.async_copy(node_hbm.at[idx_v.at[pl.ds(j * CH, CH)]],
                         rows_v.at[pl.ds(j * CH, CH)], sem)
        return carry

    def drain(j, carry):
        pltpu.make_async_copy(node_hbm.at[idx_v.at[pl.ds(j * CH, CH)]],
                              rows_v.at[pl.ds(j * CH, CH)], sem).wait()
        return carry

    lax.fori_loop(0, nch, fire, 0)
    lax.fori_loop(0, nch, drain, 0)
    pltpu.sync_copy(rows_v, x_hbm.at[wid, :, pl.ds(0, NS)])


def _gather(node_attr, edge_dst):
    epw = edge_dst.shape[0] // NW
    mesh = plsc.VectorSubcoreMesh(core_axis_name="c", subcore_axis_name="s")
    k = functools.partial(
        pl.kernel,
        out_type=jax.ShapeDtypeStruct((NW, epw, 128), jnp.float32),
        mesh=mesh,
        compiler_params=pltpu.CompilerParams(use_tc_tiling_on_sc=False),
        scratch_types=[
            pltpu.VMEM((epw,), jnp.int32),
            pltpu.VMEM((epw, NS), jnp.float32),
            pltpu.SemaphoreType.DMA,
        ],
    )(_gather_body)
    return k(node_attr, edge_dst)


def _scatter_body(tp_hbm, src_hbm, out_hbm, idx_v, vals_v, zrow_v, acc_sh,
                  isem, vsem):
    cid = lax.axis_index("c")
    sid = lax.axis_index("s")
    wid = sid * 2 + cid
    stripe = acc_sh.shape[0] // 16
    nch = idx_v.shape[0]
    base = wid * nch * CH

    def ifire(j, carry):
        pltpu.async_copy(src_hbm.at[pl.ds(base + j * CH, CH)],
                         idx_v.at[j], isem)
        return carry

    def idrain(j, carry):
        pltpu.make_async_copy(src_hbm.at[pl.ds(base + j * CH, CH)],
                              idx_v.at[j], isem).wait()
        return carry

    lax.fori_loop(0, nch, ifire, 0)

    def zb(j, carry):
        zrow_v[j, pl.ds(0, 16)] = jnp.zeros((16,), jnp.float32)
        zrow_v[j, pl.ds(16, 16)] = jnp.zeros((16,), jnp.float32)
        return carry

    lax.fori_loop(0, stripe, zb, 0)
    pltpu.sync_copy(zrow_v, acc_sh.at[pl.ds(sid * stripe, stripe)])
    lax.fori_loop(0, nch, idrain, 0)
    plsc.subcore_barrier()

    nvb = tp_hbm.shape[1]
    pltpu.async_copy(tp_hbm.at[wid, 0, :, pl.ds(0, TP_W)], vals_v.at[0], vsem)

    def body(cc, carry):
        buf = lax.rem(cc, 2)
        pltpu.make_async_copy(tp_hbm.at[wid, 0, :, pl.ds(0, TP_W)],
                              vals_v.at[0], vsem).wait()

        @pl.when(cc + 1 < nvb)
        def _():
            pltpu.async_copy(tp_hbm.at[wid, cc + 1, :, pl.ds(0, TP_W)],
                             vals_v.at[lax.rem(cc + 1, 2)], vsem)

        def inner(kk, icarry):
            pltpu.sync_copy(vals_v.at[buf, pl.ds(kk * CH, CH)],
                            acc_sh.at[idx_v.at[cc * (VB // CH) + kk]],
                            add=True)
            return icarry

        lax.fori_loop(0, VB // CH, inner, 0)
        return carry

    lax.fori_loop(0, nvb, body, 0)
    plsc.subcore_barrier()
    pltpu.sync_copy(acc_sh.at[pl.ds(sid * stripe, stripe)],
                    out_hbm.at[cid, pl.ds(sid * stripe, stripe),
                               pl.ds(0, TP_W)])


def _scatter(tp4, edge_src, n_nodes):
    epw = edge_src.shape[0] // NW
    mesh = plsc.VectorSubcoreMesh(core_axis_name="c", subcore_axis_name="s")
    k = functools.partial(
        pl.kernel,
        out_type=jax.ShapeDtypeStruct((2, n_nodes, 128), jnp.float32),
        mesh=mesh,
        compiler_params=pltpu.CompilerParams(use_tc_tiling_on_sc=False),
        scratch_types=[
            pltpu.VMEM((epw // CH, CH), jnp.int32),
            pltpu.VMEM((2, VB, TP_W), jnp.float32),
            pltpu.VMEM((n_nodes // 16, TP_W), jnp.float32),
            pltpu.VMEM_SHARED((n_nodes, TP_W), jnp.float32),
            pltpu.SemaphoreType.DMA,
            pltpu.SemaphoreType.DMA,
        ],
    )(_scatter_body)
    return k(tp4, edge_src)


def _tp_body(ea_ref, x_ref, sh_ref, w1t_ref, b1_ref, w2t_ref, b2_ref,
             r_ref, s_ref, q_ref, out_ref):
    hp = lax.Precision.DEFAULT
    c00 = (((0,), (0,)), ((), ()))
    ht = jnp.maximum(
        jnp.dot(w1t_ref[...], ea_ref[...], precision=hp,
                preferred_element_type=jnp.float32) + b1_ref[...], 0.0)
    wt = jnp.dot(w2t_ref[...], ht, precision=hp,
                 preferred_element_type=jnp.float32) + b2_ref[...]
    x16 = x_ref[:, :NS]
    xrt = lax.dot_general(r_ref[...], x16, (((0,), (1,)), ((), ())),
                          precision=hp, preferred_element_type=jnp.float32)
    a = lax.dot_general(xrt * wt, s_ref[...], c00,
                        precision=hp, preferred_element_type=jnp.float32)
    shx = lax.dot_general(sh_ref[...], q_ref[...], c00,
                          precision=hp, preferred_element_type=jnp.float32)
    lane = lax.broadcasted_iota(jnp.int32, (BE, TP_W), 1)
    tp = a * shx + jnp.where(lane == NS + NV * 3, 1.0, 0.0)
    out_ref[:, pl.ds(0, TP_W)] = tp


def _tp_edges(ea_t, xp, sh_t, W1t, b1, W2t, b2, R, S, Q):
    e = ea_t.shape[1]
    grid = e // BE
    full = lambda i: (0, 0)
    return pl.pallas_call(
        _tp_body,
        grid=(grid,),
        in_specs=[
            pl.BlockSpec((ea_t.shape[0], BE), lambda i: (0, i)),
            pl.BlockSpec((BE, 128), lambda i: (i, 0)),
            pl.BlockSpec((SH, BE), lambda i: (0, i)),
            pl.BlockSpec(W1t.shape, full),
            pl.BlockSpec((b1.shape[0], 1), full),
            pl.BlockSpec(W2t.shape, full),
            pl.BlockSpec((b2.shape[0], 1), full),
            pl.BlockSpec(R.shape, full),
            pl.BlockSpec(S.shape, full),
            pl.BlockSpec(Q.shape, full),
        ],
        out_specs=pl.BlockSpec((BE, 128), lambda i: (i, 0)),
        out_shape=jax.ShapeDtypeStruct((e, 128), jnp.float32),
    )(ea_t, xp, sh_t, W1t, b1, W2t, b2, R, S, Q)


def _combine_body(p_ref, o_ref):
    ps = p_ref[0, :, :TP_W] + p_ref[1, :, :TP_W]
    cnt = jnp.maximum(ps[:, NS + NV * 3:NS + NV * 3 + 1], 1.0)
    o_ref[...] = ps[:, :NS + NV * 3] / cnt


def _combine(partials):
    n = partials.shape[1]
    return pl.pallas_call(
        _combine_body,
        grid=(1,),
        in_specs=[pl.BlockSpec(partials.shape, lambda i: (0, 0, 0))],
        out_specs=pl.BlockSpec((n, NS + NV * 3), lambda i: (0, 0)),
        out_shape=jax.ShapeDtypeStruct((n, NS + NV * 3), jnp.float32),
    )(partials)


def kernel(node_attr, edge_attr, edge_sh, W1, b1, W2, b2, edge_index):
    n_nodes = node_attr.shape[0]
    e = edge_attr.shape[0]
    epw = e // NW
    R, S, Q = _sel_matrices()

    edge_src = edge_index[0]
    edge_dst = edge_index[1]

    x = _gather(node_attr, edge_dst).reshape(e, 128)
    tp = _tp_edges(edge_attr.T, x, edge_sh.T, W1.T, b1.reshape(-1, 1),
                   W2.T, b2.reshape(-1, 1), R, S, Q)
    partials = _scatter(tp.reshape(NW, epw // VB, VB, 128), edge_src,
                        n_nodes)
    return _combine(partials)


# speedup vs baseline: 8.4150x; 1.0630x over previous
import functools

import numpy as np
import jax
import jax.numpy as jnp
from jax import lax
from jax.experimental import pallas as pl
from jax.experimental.pallas import tpu as pltpu
from jax.experimental.pallas import tpu_sc as plsc

NS = 16
NV = 4
SH = 9
WN = NS * NS + NS * NV
TP_W = 32

NW = 32
CH = 40
VB = 1000
BE = 6400


def _sel_matrices():
    r = np.zeros((NS, WN), np.float32)
    for c in range(NS * NS):
        r[c // NS, c] = 1.0
    for c in range(NS * NV):
        r[c // NV, NS * NS + c] = 1.0
    s = np.zeros((WN, TP_W), np.float32)
    norm = 1.0 / np.sqrt(float(NS))
    for i in range(NS):
        for m in range(NS):
            s[i * NS + m, m] = norm
        for m in range(NV):
            for k in range(3):
                s[NS * NS + i * NV + m, NS + m * 3 + k] = norm
    q = np.zeros((SH, TP_W), np.float32)
    q[0, :NS] = 1.0
    for m in range(NV):
        for k in range(3):
            q[1 + k, NS + m * 3 + k] = 1.0
    return jnp.asarray(r), jnp.asarray(s), jnp.asarray(q)


def _gather_body(node_hbm, ei_hbm, x_hbm, idx_v, rows_v, sem):
    wid = lax.axis_index("s") * 2 + lax.axis_index("c")
    epw = idx_v.shape[0]
    base = wid * epw
    pltpu.sync_copy(ei_hbm.at[1, pl.ds(base, epw)], idx_v)
    nch = epw // CH

    def fire(j, carry):
        pltpu.async_copy(node_hbm.at[idx_v.at[pl.ds(j * CH, CH)]],
                         rows_v.at[pl.ds(j * CH, CH)], sem)
        return carry

    def drain(j, carry):
        pltpu.make_async_copy(node_hbm.at[idx_v.at[pl.ds(j * CH, CH)]],
                              rows_v.at[pl.ds(j * CH, CH)], sem).wait()
        return carry

    lax.fori_loop(0, nch, fire, 0)
    lax.fori_loop(0, nch, drain, 0)
    pltpu.sync_copy(rows_v, x_hbm.at[wid, :, pl.ds(0, NS)])


def _gather(node_attr, edge_index):
    epw = edge_index.shape[1] // NW
    mesh = plsc.VectorSubcoreMesh(core_axis_name="c", subcore_axis_name="s")
    k = functools.partial(
        pl.kernel,
        out_type=jax.ShapeDtypeStruct((NW, epw, 128), jnp.float32),
        mesh=mesh,
        compiler_params=pltpu.CompilerParams(use_tc_tiling_on_sc=False),
        scratch_types=[
            pltpu.VMEM((epw,), jnp.int32),
            pltpu.VMEM((epw, NS), jnp.float32),
            pltpu.SemaphoreType.DMA,
        ],
    )(_gather_body)
    return k(node_attr, edge_index)


def _scatter_body(tp_hbm, ei_hbm, out_hbm, idx_v, vals_v, zrow_v, acc_sh,
                  isem, vsem, ssem):
    cid = lax.axis_index("c")
    sid = lax.axis_index("s")
    wid = sid * 2 + cid
    stripe = acc_sh.shape[0] // 16
    nch = idx_v.shape[0]
    base = wid * nch * CH

    def ifire(j, carry):
        pltpu.async_copy(ei_hbm.at[0, pl.ds(base + j * CH, CH)],
                         idx_v.at[j], isem)
        return carry

    def idrain(j, carry):
        pltpu.make_async_copy(ei_hbm.at[0, pl.ds(base + j * CH, CH)],
                              idx_v.at[j], isem).wait()
        return carry

    lax.fori_loop(0, nch, ifire, 0)

    def zb(j, carry):
        zrow_v[j, pl.ds(0, 16)] = jnp.zeros((16,), jnp.float32)
        zrow_v[j, pl.ds(16, 16)] = jnp.zeros((16,), jnp.float32)
        return carry

    lax.fori_loop(0, stripe, zb, 0)
    pltpu.sync_copy(zrow_v, acc_sh.at[pl.ds(sid * stripe, stripe)])
    lax.fori_loop(0, nch, idrain, 0)
    plsc.subcore_barrier()

    nvb = tp_hbm.shape[1]
    pltpu.async_copy(tp_hbm.at[wid, 0, :, pl.ds(0, TP_W)], vals_v.at[0], vsem)

    def body(cc, carry):
        buf = lax.rem(cc, 2)
        pltpu.make_async_copy(tp_hbm.at[wid, 0, :, pl.ds(0, TP_W)],
                              vals_v.at[0], vsem).wait()

        @pl.when(cc + 1 < nvb)
        def _():
            pltpu.async_copy(tp_hbm.at[wid, cc + 1, :, pl.ds(0, TP_W)],
                             vals_v.at[lax.rem(cc + 1, 2)], vsem)

        def ifire2(kk, icarry):
            pltpu.make_async_copy(
                vals_v.at[buf, pl.ds(kk * CH, CH)],
                acc_sh.at[idx_v.at[cc * (VB // CH) + kk]],
                ssem).start(add=True)
            return icarry

        def idrain2(kk, icarry):
            pltpu.make_async_copy(
                vals_v.at[buf, pl.ds(kk * CH, CH)],
                acc_sh.at[idx_v.at[cc * (VB // CH) + kk]],
                ssem).wait()
            return icarry

        lax.fori_loop(0, VB // CH, ifire2, 0)
        lax.fori_loop(0, VB // CH, idrain2, 0)
        return carry

    lax.fori_loop(0, nvb, body, 0)
    plsc.subcore_barrier()
    pltpu.sync_copy(acc_sh.at[pl.ds(sid * stripe, stripe)],
                    out_hbm.at[cid, pl.ds(sid * stripe, stripe),
                               pl.ds(0, TP_W)])


def _scatter(tp4, edge_index, n_nodes):
    epw = edge_index.shape[1] // NW
    mesh = plsc.VectorSubcoreMesh(core_axis_name="c", subcore_axis_name="s")
    k = functools.partial(
        pl.kernel,
        out_type=jax.ShapeDtypeStruct((2, n_nodes, 128), jnp.float32),
        mesh=mesh,
        compiler_params=pltpu.CompilerParams(use_tc_tiling_on_sc=False),
        scratch_types=[
            pltpu.VMEM((epw // CH, CH), jnp.int32),
            pltpu.VMEM((2, VB, TP_W), jnp.float32),
            pltpu.VMEM((n_nodes // 16, TP_W), jnp.float32),
            pltpu.VMEM_SHARED((n_nodes, TP_W), jnp.float32),
            pltpu.SemaphoreType.DMA,
            pltpu.SemaphoreType.DMA,
            pltpu.SemaphoreType.DMA,
        ],
    )(_scatter_body)
    return k(tp4, edge_index)


def _tp_body(ea_ref, x_ref, sh_ref, w1t_ref, b1_ref, w2t_ref, b2_ref,
             r_ref, s_ref, q_ref, out_ref):
    hp = lax.Precision.DEFAULT
    c00 = (((0,), (0,)), ((), ()))
    ht = jnp.maximum(
        jnp.dot(w1t_ref[...], ea_ref[...], precision=hp,
                preferred_element_type=jnp.float32) + b1_ref[...], 0.0)
    wt = jnp.dot(w2t_ref[...], ht, precision=hp,
                 preferred_element_type=jnp.float32) + b2_ref[...]
    x16 = x_ref[:, :NS]
    xrt = lax.dot_general(r_ref[...], x16, (((0,), (1,)), ((), ())),
                          precision=hp, preferred_element_type=jnp.float32)
    a = lax.dot_general(xrt * wt, s_ref[...], c00,
                        precision=hp, preferred_element_type=jnp.float32)
    shx = lax.dot_general(sh_ref[...], q_ref[...], c00,
                          precision=hp, preferred_element_type=jnp.float32)
    lane = lax.broadcasted_iota(jnp.int32, (BE, TP_W), 1)
    tp = a * shx + jnp.where(lane == NS + NV * 3, 1.0, 0.0)
    out_ref[:, pl.ds(0, TP_W)] = tp


def _tp_edges(ea_t, xp, sh_t, W1t, b1, W2t, b2, R, S, Q):
    e = ea_t.shape[1]
    grid = e // BE
    full = lambda i: (0, 0)
    return pl.pallas_call(
        _tp_body,
        grid=(grid,),
        in_specs=[
            pl.BlockSpec((ea_t.shape[0], BE), lambda i: (0, i)),
            pl.BlockSpec((BE, 128), lambda i: (i, 0)),
            pl.BlockSpec((SH, BE), lambda i: (0, i)),
            pl.BlockSpec(W1t.shape, full),
            pl.BlockSpec((b1.shape[0], 1), full),
            pl.BlockSpec(W2t.shape, full),
            pl.BlockSpec((b2.shape[0], 1), full),
            pl.BlockSpec(R.shape, full),
            pl.BlockSpec(S.shape, full),
            pl.BlockSpec(Q.shape, full),
        ],
        out_specs=pl.BlockSpec((BE, 128), lambda i: (i, 0)),
        out_shape=jax.ShapeDtypeStruct((e, 128), jnp.float32),
    )(ea_t, xp, sh_t, W1t, b1, W2t, b2, R, S, Q)


def _combine_body(p_ref, m_ref, o_ref):
    ps = p_ref[0, :, :TP_W] + p_ref[1, :, :TP_W]
    st = lax.dot_general(m_ref[...], ps, (((0,), (1,)), ((), ())),
                         precision=lax.Precision.HIGHEST,
                         preferred_element_type=jnp.float32)
    nc = NS + NV * 3
    cnt = jnp.maximum(st[nc:nc + 1, :], 1.0)
    o_ref[...] = st[:nc, :] / cnt


def _combine(partials, M):
    n = partials.shape[1]
    return pl.pallas_call(
        _combine_body,
        grid=(1,),
        in_specs=[pl.BlockSpec(partials.shape, lambda i: (0, 0, 0)),
                  pl.BlockSpec(M.shape, lambda i: (0, 0))],
        out_specs=pl.BlockSpec((NS + NV * 3, n), lambda i: (0, 0)),
        out_shape=jax.ShapeDtypeStruct((NS + NV * 3, n), jnp.float32),
    )(partials, M)


def kernel(node_attr, edge_attr, edge_sh, W1, b1, W2, b2, edge_index):
    n_nodes = node_attr.shape[0]
    e = edge_attr.shape[0]
    epw = e // NW
    R, S, Q = _sel_matrices()

    M = jnp.asarray(np.eye(TP_W, NS + NV * 3 + 1, dtype=np.float32))

    x = _gather(node_attr, edge_index).reshape(e, 128)
    tp = _tp_edges(edge_attr.T, x, edge_sh.T, W1.T, b1.reshape(-1, 1),
                   W2.T, b2.reshape(-1, 1), R, S, Q)
    partials = _scatter(tp.reshape(NW, epw // VB, VB, 128), edge_index,
                        n_nodes)
    return _combine(partials, M).T
